# baseline (speedup 1.0000x reference)
import jax
import jax.numpy as jnp
from jax import lax
from jax.experimental import pallas as pl
from jax.experimental.pallas import tpu as pltpu
from jax.experimental.pallas import tpu_sc as plsc

_B, _S, _V = 1024, 50, 999
_NC, _NS, _L = 2, 16, 16
_NW = _NC * _NS
_XPW = _B // _NW
_XB = 2
_NBATCH = _XPW // _XB
_IPB = _XB * _S
_IDX_GROUPS = (_IPB + _L - 1) // _L
_IDX_PAD = _NBATCH * _IPB + _L


def _scatter_batch(buf, idx_v, b, value):
    for g in range(_IDX_GROUPS):
        base = g * _L
        base_row = base // _S
        thr = (base_row + 1) * _S
        k = lax.broadcasted_iota(jnp.int32, (_L,), 0) + base
        v = idx_v[pl.ds(b * _IPB + base, _L)]
        xrow = jnp.where(k >= thr, base_row + 1, base_row)
        seg = k - xrow * _S
        col = v - 1
        mask = (k < _IPB) & (v >= 1)
        plsc.store_scatter(buf, [xrow, seg, col],
                           jnp.full((_L,), value, jnp.float32), mask=mask)


def _body(x_hbm, out_hbm, idx_v, buf, sem):
    del sem
    wid = lax.axis_index("s") * _NC + lax.axis_index("c")

    pltpu.sync_copy(x_hbm.at[pl.ds(wid * _XPW * _S, _XPW * _S)],
                    idx_v.at[pl.ds(0, _XPW * _S)])

    zeros = jnp.zeros((_L,), jnp.float32)

    def _zero_row(r, _):
        xr = jnp.where(r >= _S, 1, 0)
        sg = r - xr * _S
        for c in range(0, _V, _L):
            buf[xr, sg, pl.ds(min(c, _V - _L), _L)] = zeros
        return 0

    lax.fori_loop(0, _XB * _S, _zero_row, 0)

    def _batch(b, _):
        _scatter_batch(buf, idx_v, b, 1.0)
        pltpu.sync_copy(buf, out_hbm.at[pl.ds(wid * _XPW + b * _XB, _XB)])
        _scatter_batch(buf, idx_v, b, 0.0)
        return 0

    lax.fori_loop(0, _NBATCH, _batch, 0)


def kernel(x):
    xf = x.reshape(-1)
    mesh = plsc.VectorSubcoreMesh(core_axis_name="c", subcore_axis_name="s",
                                  num_cores=_NC, num_subcores=_NS)
    return pl.kernel(
        _body,
        out_type=jax.ShapeDtypeStruct((_B, _S, _V), jnp.float32),
        mesh=mesh,
        compiler_params=pltpu.CompilerParams(needs_layout_passes=False),
        scratch_types=[
            pltpu.VMEM((_IDX_PAD,), jnp.int32),
            pltpu.VMEM((_XB, _S, _V), jnp.float32),
            pltpu.SemaphoreType.DMA,
        ],
    )(xf)


# speedup vs baseline: 1.0066x; 1.0066x over previous
import jax
import jax.numpy as jnp
from jax import lax
from jax.experimental import pallas as pl
from jax.experimental.pallas import tpu as pltpu
from jax.experimental.pallas import tpu_sc as plsc

_B, _S, _V = 1024, 50, 999
_NC, _NS, _L = 2, 16, 16
_NW = _NC * _NS
_XPW = _B // _NW
_IPB = _S
_IDX_GROUPS = (_IPB + _L - 1) // _L
_NBATCH = _XPW
_IDX_PAD = _NBATCH * _IPB + _L


def _scatter_batch(buf, idx_v, b, value):
    for g in range(_IDX_GROUPS):
        base = g * _L
        k = lax.broadcasted_iota(jnp.int32, (_L,), 0) + base
        v = idx_v[pl.ds(b * _IPB + base, _L)]
        zero = jnp.zeros((_L,), jnp.int32)
        col = v - 1
        mask = (k < _IPB) & (v >= 1)
        plsc.store_scatter(buf, [zero, k, col],
                           jnp.full((_L,), value, jnp.float32), mask=mask)


def _body(x_hbm, out_hbm, idx_v, buf0, buf1, sem0, sem1):
    wid = lax.axis_index("s") * _NC + lax.axis_index("c")
    row0 = wid * _XPW

    pltpu.sync_copy(x_hbm.at[pl.ds(row0 * _S, _XPW * _S)],
                    idx_v.at[pl.ds(0, _XPW * _S)])

    zeros = jnp.zeros((_L,), jnp.float32)

    def _zero_seg(r, _):
        for c in range(0, _V, _L):
            buf0[0, r, pl.ds(min(c, _V - _L), _L)] = zeros
            buf1[0, r, pl.ds(min(c, _V - _L), _L)] = zeros
        return 0

    lax.fori_loop(0, _S, _zero_seg, 0)

    def _step(buf, sem, b):
        @pl.when(b >= 2)
        def _():
            pltpu.make_async_copy(
                buf, out_hbm.at[pl.ds(row0 + b - 2, 1)], sem).wait()
            _scatter_batch(buf, idx_v, b - 2, 0.0)

        _scatter_batch(buf, idx_v, b, 1.0)
        pltpu.make_async_copy(
            buf, out_hbm.at[pl.ds(row0 + b, 1)], sem).start()

    def _loop(g, _):
        _step(buf0, sem0, 2 * g)
        _step(buf1, sem1, 2 * g + 1)
        return 0

    lax.fori_loop(0, _NBATCH // 2, _loop, 0)

    pltpu.make_async_copy(
        buf0, out_hbm.at[pl.ds(row0 + _NBATCH - 2, 1)], sem0).wait()
    pltpu.make_async_copy(
        buf1, out_hbm.at[pl.ds(row0 + _NBATCH - 1, 1)], sem1).wait()


def kernel(x):
    xf = x.reshape(-1)
    mesh = plsc.VectorSubcoreMesh(core_axis_name="c", subcore_axis_name="s",
                                  num_cores=_NC, num_subcores=_NS)
    return pl.kernel(
        _body,
        out_type=jax.ShapeDtypeStruct((_B, _S, _V), jnp.float32),
        mesh=mesh,
        compiler_params=pltpu.CompilerParams(needs_layout_passes=False),
        scratch_types=[
            pltpu.VMEM((_IDX_PAD,), jnp.int32),
            pltpu.VMEM((1, _S, _V), jnp.float32),
            pltpu.VMEM((1, _S, _V), jnp.float32),
            pltpu.SemaphoreType.DMA,
            pltpu.SemaphoreType.DMA,
        ],
    )(xf)
